# baseline (speedup 1.0000x reference)
import jax
import jax.numpy as jnp
from jax.experimental import pallas as pl
from jax.experimental.pallas import tpu as pltpu


def _fused_kernel(x_ref, w_ref, beta_ref, n_ref,
                  xbuf_ref, csq_ref, rsq_ref, bbuf_ref, nbuf_ref,
                  bsem, nsem):
    i = pl.program_id(0)
    num = pl.num_programs(0)
    nb = num // 2
    rb = x_ref.shape[0]

    @pl.when(i == 0)
    def _init():
        csq_ref[...] = jnp.zeros_like(csq_ref)
        w = w_ref[...]
        rsq_ref[...] = jnp.sum(w * w, axis=1)[None, :]

    @pl.when(i < nb)
    def _phase_a():
        xb = x_ref[...]
        xbuf_ref[pl.ds(i * rb, rb), :] = xb
        xsq = xb * xb
        part = xsq.reshape(8, rb // 8, xsq.shape[1]).sum(axis=0)
        csq_ref[...] += part.sum(axis=0, keepdims=True)

    @pl.when(i >= nb)
    def _phase_b():
        j = i - nb
        cinv = jax.lax.rsqrt(csq_ref[...])
        rinv = jax.lax.rsqrt(rsq_ref[...])
        xb = xbuf_ref[pl.ds(j * rb, rb), :] * cinv
        beta = jax.lax.dot_general(
            xb, w_ref[...],
            dimension_numbers=(((1,), (1,)), ((), ())),
            preferred_element_type=jnp.float32,
        ) * rinv
        bbuf_ref[pl.ds(j * rb, rb), :] = beta
        pltpu.make_async_copy(
            bbuf_ref.at[pl.ds(j * rb, rb), :],
            beta_ref.at[pl.ds(j * rb, rb), :],
            bsem.at[j],
        ).start()
        nbuf_ref[pl.ds(j * rb, rb)] = jnp.argmax(beta, axis=1).astype(jnp.int32)

    @pl.when(i == num - 1)
    def _drain():
        pltpu.make_async_copy(nbuf_ref, n_ref, nsem).start()
        for j2 in range(_NB):
            pltpu.make_async_copy(
                bbuf_ref.at[pl.ds(j2 * rb, rb), :],
                beta_ref.at[pl.ds(j2 * rb, rb), :],
                bsem.at[j2],
            ).wait()
        pltpu.make_async_copy(nbuf_ref, n_ref, nsem).wait()


_RB = 512
_NB = 8192 // _RB


def kernel(all_ts, clustering_flag, W):
    del clustering_flag
    B = all_ts.shape[0]
    x = all_ts.reshape(B, -1).astype(W.dtype)
    F = x.shape[1]
    N = W.shape[0]
    nb = B // _RB

    beta, n_star = pl.pallas_call(
        _fused_kernel,
        grid=(2 * nb,),
        in_specs=[
            pl.BlockSpec((_RB, F), lambda i: (jnp.minimum(i, nb - 1), 0)),
            pl.BlockSpec((N, F), lambda i: (0, 0)),
        ],
        out_specs=[
            pl.BlockSpec(memory_space=pl.ANY),
            pl.BlockSpec(memory_space=pl.ANY),
        ],
        out_shape=[
            jax.ShapeDtypeStruct((B, N), jnp.float32),
            jax.ShapeDtypeStruct((B,), jnp.int32),
        ],
        scratch_shapes=[
            pltpu.VMEM((B, F), jnp.float32),
            pltpu.VMEM((1, F), jnp.float32),
            pltpu.VMEM((1, N), jnp.float32),
            pltpu.VMEM((B, N), jnp.float32),
            pltpu.VMEM((B,), jnp.int32),
            pltpu.SemaphoreType.DMA((nb,)),
            pltpu.SemaphoreType.DMA,
        ],
    )(x, W)

    indices = jnp.arange(B, dtype=jnp.int32)
    return n_star, indices, beta


# speedup vs baseline: 1.1415x; 1.1415x over previous
import jax
import jax.numpy as jnp
from jax.experimental import pallas as pl
from jax.experimental.pallas import tpu as pltpu


def _fused_kernel(x_ref, w_ref, beta_ref, n_ref,
                  xbuf_ref, csq_ref, rsq_ref, bbuf_ref, nbuf_ref,
                  bsem, nsem):
    i = pl.program_id(0)
    num = pl.num_programs(0)
    nb = num // 2
    rb = x_ref.shape[0]

    @pl.when(i == 0)
    def _init():
        csq_ref[...] = jnp.zeros_like(csq_ref)
        w = w_ref[...]
        rsq_ref[...] = jnp.sum(w * w, axis=1)[None, :]

    @pl.when(i < nb)
    def _phase_a():
        xb = x_ref[...]
        xbuf_ref[pl.ds(i * rb, rb), :] = xb
        xsq = xb * xb
        part = xsq.reshape(8, rb // 8, xsq.shape[1]).sum(axis=0)
        csq_ref[...] += part.sum(axis=0, keepdims=True)

    @pl.when(i >= nb)
    def _phase_b():
        j = i - nb
        cinv = jax.lax.rsqrt(csq_ref[...])
        rinv = jax.lax.rsqrt(rsq_ref[...])
        xb = xbuf_ref[pl.ds(j * rb, rb), :] * cinv
        beta = jax.lax.dot_general(
            xb, w_ref[...],
            dimension_numbers=(((1,), (1,)), ((), ())),
            preferred_element_type=jnp.float32,
        ) * rinv
        bbuf_ref[pl.ds(j * rb, rb), :] = beta
        pltpu.make_async_copy(
            bbuf_ref.at[pl.ds(j * rb, rb), :],
            beta_ref.at[pl.ds(j * rb, rb), :],
            bsem.at[j],
        ).start()
        nbuf_ref[pl.ds(j * rb, rb)] = jnp.argmax(beta, axis=1).astype(jnp.int32)

    @pl.when(i == num - 1)
    def _drain():
        pltpu.make_async_copy(nbuf_ref, n_ref, nsem).start()
        for j2 in range(_NB):
            pltpu.make_async_copy(
                bbuf_ref.at[pl.ds(j2 * rb, rb), :],
                beta_ref.at[pl.ds(j2 * rb, rb), :],
                bsem.at[j2],
            ).wait()
        pltpu.make_async_copy(nbuf_ref, n_ref, nsem).wait()


_RB = 1024
_NB = 8192 // _RB


def kernel(all_ts, clustering_flag, W):
    del clustering_flag
    B = all_ts.shape[0]
    x = all_ts.reshape(B, -1).astype(W.dtype)
    F = x.shape[1]
    N = W.shape[0]
    nb = B // _RB

    beta, n_star = pl.pallas_call(
        _fused_kernel,
        grid=(2 * nb,),
        in_specs=[
            pl.BlockSpec((_RB, F), lambda i: (jnp.minimum(i, nb - 1), 0)),
            pl.BlockSpec((N, F), lambda i: (0, 0)),
        ],
        out_specs=[
            pl.BlockSpec(memory_space=pl.ANY),
            pl.BlockSpec(memory_space=pl.ANY),
        ],
        out_shape=[
            jax.ShapeDtypeStruct((B, N), jnp.float32),
            jax.ShapeDtypeStruct((B,), jnp.int32),
        ],
        scratch_shapes=[
            pltpu.VMEM((B, F), jnp.float32),
            pltpu.VMEM((1, F), jnp.float32),
            pltpu.VMEM((1, N), jnp.float32),
            pltpu.VMEM((B, N), jnp.float32),
            pltpu.VMEM((B,), jnp.int32),
            pltpu.SemaphoreType.DMA((nb,)),
            pltpu.SemaphoreType.DMA,
        ],
    )(x, W)

    indices = jnp.arange(B, dtype=jnp.int32)
    return n_star, indices, beta


# speedup vs baseline: 1.1591x; 1.0154x over previous
import jax
import jax.numpy as jnp
from jax.experimental import pallas as pl
from jax.experimental.pallas import tpu as pltpu


def _fused_kernel(x_ref, w_ref, beta_ref, n_ref,
                  xbuf_ref, csq_ref, rsq_ref, bbuf_ref, nbuf_ref,
                  bsem, nsem):
    i = pl.program_id(0)
    num = pl.num_programs(0)
    nb = num // 2
    rb = x_ref.shape[0]

    @pl.when(i == 0)
    def _init():
        csq_ref[...] = jnp.zeros_like(csq_ref)
        w = w_ref[...]
        rsq_ref[...] = jnp.sum(w * w, axis=1)[None, :]

    @pl.when(i < nb)
    def _phase_a():
        xb = x_ref[...]
        xbuf_ref[pl.ds(i * rb, rb), :] = xb
        xsq = xb * xb
        part = xsq.reshape(8, rb // 8, xsq.shape[1]).sum(axis=0)
        csq_ref[...] += part.sum(axis=0, keepdims=True)

    @pl.when(i >= nb)
    def _phase_b():
        j = i - nb
        cinv = jax.lax.rsqrt(csq_ref[...])
        rinv = jax.lax.rsqrt(rsq_ref[...])
        xb = xbuf_ref[pl.ds(j * rb, rb), :] * cinv
        beta = jax.lax.dot_general(
            xb, w_ref[...],
            dimension_numbers=(((1,), (1,)), ((), ())),
            preferred_element_type=jnp.float32,
        ) * rinv
        bbuf_ref[pl.ds(j * rb, rb), :] = beta
        pltpu.make_async_copy(
            bbuf_ref.at[pl.ds(j * rb, rb), :],
            beta_ref.at[pl.ds(j * rb, rb), :],
            bsem.at[j],
        ).start()
        nbuf_ref[pl.ds(j * rb, rb)] = jnp.argmax(beta, axis=1).astype(jnp.int32)

    @pl.when(i == num - 1)
    def _drain():
        pltpu.make_async_copy(nbuf_ref, n_ref, nsem).start()
        for j2 in range(_NB):
            pltpu.make_async_copy(
                bbuf_ref.at[pl.ds(j2 * rb, rb), :],
                beta_ref.at[pl.ds(j2 * rb, rb), :],
                bsem.at[j2],
            ).wait()
        pltpu.make_async_copy(nbuf_ref, n_ref, nsem).wait()


_RB = 2048
_NB = 8192 // _RB


def kernel(all_ts, clustering_flag, W):
    del clustering_flag
    B = all_ts.shape[0]
    x = all_ts.reshape(B, -1).astype(W.dtype)
    F = x.shape[1]
    N = W.shape[0]
    nb = B // _RB

    beta, n_star = pl.pallas_call(
        _fused_kernel,
        grid=(2 * nb,),
        in_specs=[
            pl.BlockSpec((_RB, F), lambda i: (jnp.minimum(i, nb - 1), 0)),
            pl.BlockSpec((N, F), lambda i: (0, 0)),
        ],
        out_specs=[
            pl.BlockSpec(memory_space=pl.ANY),
            pl.BlockSpec(memory_space=pl.ANY),
        ],
        out_shape=[
            jax.ShapeDtypeStruct((B, N), jnp.float32),
            jax.ShapeDtypeStruct((B,), jnp.int32),
        ],
        scratch_shapes=[
            pltpu.VMEM((B, F), jnp.float32),
            pltpu.VMEM((1, F), jnp.float32),
            pltpu.VMEM((1, N), jnp.float32),
            pltpu.VMEM((B, N), jnp.float32),
            pltpu.VMEM((B,), jnp.int32),
            pltpu.SemaphoreType.DMA((nb,)),
            pltpu.SemaphoreType.DMA,
        ],
    )(x, W)

    indices = jnp.arange(B, dtype=jnp.int32)
    return n_star, indices, beta


# speedup vs baseline: 1.2322x; 1.0630x over previous
import jax
import jax.numpy as jnp
from jax.experimental import pallas as pl
from jax.experimental.pallas import tpu as pltpu

_RB_A = 1024
_NA = 8192 // _RB_A
_ST = 512
_NT = 8192 // _ST


def _fused_kernel(x_ref, w_ref, beta_ref, n_ref,
                  xbuf_ref, csq_ref, rsq_ref, bbuf_ref, nbuf_ref,
                  bsem, nsem):
    i = pl.program_id(0)
    rb = x_ref.shape[0]

    @pl.when(i == 0)
    def _init():
        csq_ref[...] = jnp.zeros_like(csq_ref)
        w = w_ref[...]
        rsq_ref[...] = jnp.sum(w * w, axis=1)[None, :]

    @pl.when(i < _NA)
    def _phase_a():
        xb = x_ref[...]
        xbuf_ref[pl.ds(i * rb, rb), :] = xb
        xsq = xb * xb
        part = xsq.reshape(8, rb // 8, xsq.shape[1]).sum(axis=0)
        csq_ref[...] += part.sum(axis=0, keepdims=True)

    @pl.when(i == _NA)
    def _phase_b():
        cinv = jax.lax.rsqrt(csq_ref[...])
        rinv = jax.lax.rsqrt(rsq_ref[...])
        w = w_ref[...]
        for t in range(_NT):
            sl = pl.ds(t * _ST, _ST)
            xb = xbuf_ref[sl, :] * cinv
            beta = jax.lax.dot_general(
                xb, w,
                dimension_numbers=(((1,), (1,)), ((), ())),
                preferred_element_type=jnp.float32,
            ) * rinv
            bbuf_ref[sl, :] = beta
            pltpu.make_async_copy(
                bbuf_ref.at[sl, :], beta_ref.at[sl, :], bsem.at[t],
            ).start()
            nbuf_ref[sl] = jnp.argmax(beta, axis=1).astype(jnp.int32)
        pltpu.make_async_copy(nbuf_ref, n_ref, nsem).start()
        for t in range(_NT):
            sl = pl.ds(t * _ST, _ST)
            pltpu.make_async_copy(
                bbuf_ref.at[sl, :], beta_ref.at[sl, :], bsem.at[t],
            ).wait()
        pltpu.make_async_copy(nbuf_ref, n_ref, nsem).wait()


def kernel(all_ts, clustering_flag, W):
    del clustering_flag
    B = all_ts.shape[0]
    x = all_ts.reshape(B, -1).astype(W.dtype)
    F = x.shape[1]
    N = W.shape[0]

    beta, n_star = pl.pallas_call(
        _fused_kernel,
        grid=(_NA + 1,),
        in_specs=[
            pl.BlockSpec((_RB_A, F), lambda i: (jnp.minimum(i, _NA - 1), 0)),
            pl.BlockSpec((N, F), lambda i: (0, 0)),
        ],
        out_specs=[
            pl.BlockSpec(memory_space=pl.ANY),
            pl.BlockSpec(memory_space=pl.ANY),
        ],
        out_shape=[
            jax.ShapeDtypeStruct((B, N), jnp.float32),
            jax.ShapeDtypeStruct((B,), jnp.int32),
        ],
        scratch_shapes=[
            pltpu.VMEM((B, F), jnp.float32),
            pltpu.VMEM((1, F), jnp.float32),
            pltpu.VMEM((1, N), jnp.float32),
            pltpu.VMEM((B, N), jnp.float32),
            pltpu.VMEM((B,), jnp.int32),
            pltpu.SemaphoreType.DMA((_NT,)),
            pltpu.SemaphoreType.DMA,
        ],
    )(x, W)

    indices = jnp.arange(B, dtype=jnp.int32)
    return n_star, indices, beta


# speedup vs baseline: 1.3251x; 1.0754x over previous
import jax
import jax.numpy as jnp
from jax.experimental import pallas as pl
from jax.experimental.pallas import tpu as pltpu

_CA = 1024
_NC = 8192 // _CA
_ST = 512
_NT = 8192 // _ST


def _fused_kernel(x_ref, w_ref, beta_ref, n_ref,
                  xbuf_ref, bbuf_ref, nbuf_ref,
                  xsem, bsem, nsem):
    for c in range(_NC):
        sl = pl.ds(c * _CA, _CA)
        pltpu.make_async_copy(
            x_ref.at[sl, :], xbuf_ref.at[sl, :], xsem.at[c],
        ).start()

    w = w_ref[...]
    rsq = jnp.sum(w * w, axis=1)[None, :]
    rinv = jax.lax.rsqrt(rsq)

    csq = jnp.zeros((1, w.shape[1]), dtype=jnp.float32)
    for c in range(_NC):
        sl = pl.ds(c * _CA, _CA)
        pltpu.make_async_copy(
            x_ref.at[sl, :], xbuf_ref.at[sl, :], xsem.at[c],
        ).wait()
        xb = xbuf_ref[sl, :]
        xsq = xb * xb
        part = xsq.reshape(8, _CA // 8, xsq.shape[1]).sum(axis=0)
        csq = csq + part.sum(axis=0, keepdims=True)
    cinv = jax.lax.rsqrt(csq)

    for t in range(_NT):
        sl = pl.ds(t * _ST, _ST)
        xb = xbuf_ref[sl, :] * cinv
        beta = jax.lax.dot_general(
            xb, w,
            dimension_numbers=(((1,), (1,)), ((), ())),
            preferred_element_type=jnp.float32,
        ) * rinv
        bbuf_ref[sl, :] = beta
        pltpu.make_async_copy(
            bbuf_ref.at[sl, :], beta_ref.at[sl, :], bsem.at[t],
        ).start()
        nbuf_ref[sl] = jnp.argmax(beta, axis=1).astype(jnp.int32)

    pltpu.make_async_copy(nbuf_ref, n_ref, nsem).start()
    for t in range(_NT):
        sl = pl.ds(t * _ST, _ST)
        pltpu.make_async_copy(
            bbuf_ref.at[sl, :], beta_ref.at[sl, :], bsem.at[t],
        ).wait()
    pltpu.make_async_copy(nbuf_ref, n_ref, nsem).wait()


def kernel(all_ts, clustering_flag, W):
    del clustering_flag
    B = all_ts.shape[0]
    x = all_ts.reshape(B, -1).astype(W.dtype)
    F = x.shape[1]
    N = W.shape[0]

    beta, n_star = pl.pallas_call(
        _fused_kernel,
        in_specs=[
            pl.BlockSpec(memory_space=pl.ANY),
            pl.BlockSpec((N, F), lambda: (0, 0)),
        ],
        out_specs=[
            pl.BlockSpec(memory_space=pl.ANY),
            pl.BlockSpec(memory_space=pl.ANY),
        ],
        out_shape=[
            jax.ShapeDtypeStruct((B, N), jnp.float32),
            jax.ShapeDtypeStruct((B,), jnp.int32),
        ],
        scratch_shapes=[
            pltpu.VMEM((B, F), jnp.float32),
            pltpu.VMEM((B, N), jnp.float32),
            pltpu.VMEM((B,), jnp.int32),
            pltpu.SemaphoreType.DMA((_NC,)),
            pltpu.SemaphoreType.DMA((_NT,)),
            pltpu.SemaphoreType.DMA,
        ],
    )(x, W)

    indices = jnp.arange(B, dtype=jnp.int32)
    return n_star, indices, beta


# speedup vs baseline: 1.3497x; 1.0186x over previous
import jax
import jax.numpy as jnp
from jax.experimental import pallas as pl
from jax.experimental.pallas import tpu as pltpu

_CA = 2048
_NC = 8192 // _CA
_ST = 512
_NT = 8192 // _ST


def _fused_kernel(x_ref, w_ref, beta_ref, n_ref,
                  xbuf_ref, wbuf_ref, bbuf_ref, nbuf_ref,
                  xsem, wsem, bsem, nsem):
    wcopy = pltpu.make_async_copy(w_ref, wbuf_ref, wsem)
    wcopy.start()
    for c in range(_NC):
        sl = pl.ds(c * _CA, _CA)
        pltpu.make_async_copy(
            x_ref.at[sl, :], xbuf_ref.at[sl, :], xsem.at[c],
        ).start()

    wcopy.wait()
    w = wbuf_ref[...]
    rsq = jnp.sum(w * w, axis=1)[None, :]
    rinv = jax.lax.rsqrt(rsq)

    csq = jnp.zeros((1, w.shape[1]), dtype=jnp.float32)
    for c in range(_NC):
        sl = pl.ds(c * _CA, _CA)
        pltpu.make_async_copy(
            x_ref.at[sl, :], xbuf_ref.at[sl, :], xsem.at[c],
        ).wait()
        xb = xbuf_ref[sl, :]
        xsq = xb * xb
        part = xsq.reshape(8, _CA // 8, xsq.shape[1]).sum(axis=0)
        csq = csq + part.sum(axis=0, keepdims=True)
    cinv = jax.lax.rsqrt(csq)

    for t in range(_NT):
        sl = pl.ds(t * _ST, _ST)
        xb = xbuf_ref[sl, :] * cinv
        beta = jax.lax.dot_general(
            xb, w,
            dimension_numbers=(((1,), (1,)), ((), ())),
            preferred_element_type=jnp.float32,
        ) * rinv
        bbuf_ref[sl, :] = beta
        pltpu.make_async_copy(
            bbuf_ref.at[sl, :], beta_ref.at[sl, :], bsem.at[t],
        ).start()
        nbuf_ref[sl] = jnp.argmax(beta, axis=1).astype(jnp.int32)

    pltpu.make_async_copy(nbuf_ref, n_ref, nsem).start()
    for t in range(_NT):
        sl = pl.ds(t * _ST, _ST)
        pltpu.make_async_copy(
            bbuf_ref.at[sl, :], beta_ref.at[sl, :], bsem.at[t],
        ).wait()
    pltpu.make_async_copy(nbuf_ref, n_ref, nsem).wait()


def kernel(all_ts, clustering_flag, W):
    del clustering_flag
    B = all_ts.shape[0]
    x = all_ts.reshape(B, -1).astype(W.dtype)
    F = x.shape[1]
    N = W.shape[0]

    beta, n_star = pl.pallas_call(
        _fused_kernel,
        in_specs=[
            pl.BlockSpec(memory_space=pl.ANY),
            pl.BlockSpec(memory_space=pl.ANY),
        ],
        out_specs=[
            pl.BlockSpec(memory_space=pl.ANY),
            pl.BlockSpec(memory_space=pl.ANY),
        ],
        out_shape=[
            jax.ShapeDtypeStruct((B, N), jnp.float32),
            jax.ShapeDtypeStruct((B,), jnp.int32),
        ],
        scratch_shapes=[
            pltpu.VMEM((B, F), jnp.float32),
            pltpu.VMEM((N, F), jnp.float32),
            pltpu.VMEM((B, N), jnp.float32),
            pltpu.VMEM((B,), jnp.int32),
            pltpu.SemaphoreType.DMA((_NC,)),
            pltpu.SemaphoreType.DMA,
            pltpu.SemaphoreType.DMA((_NT,)),
            pltpu.SemaphoreType.DMA,
        ],
    )(x, W)

    indices = jnp.arange(B, dtype=jnp.int32)
    return n_star, indices, beta
